# baseline (speedup 1.0000x reference)
import jax
import jax.numpy as jnp
from jax.experimental import pallas as pl
from jax.experimental.pallas import tpu as pltpu

NUM_PAGES = 2048
PAGE_SIZE = 64
KV_DIM = 256
MAX_SEQS = 64
PAGES_PER_SEQ = 64
MAX_SEQ_LEN = 4096

_NCHUNKS = 16
_PAGES_PER_CHUNK = NUM_PAGES // _NCHUNKS


def _body(meta_s, seq_lens_s, pi_s, pi_v, tokens_v, cache_a,
          cache_out_a, tokens_out_v, seq_lens_out_s, pi_out_v, sems):
    copies = []
    for c in range(_NCHUNKS):
        cp = pltpu.make_async_copy(
            cache_a.at[pl.ds(c * _PAGES_PER_CHUNK, _PAGES_PER_CHUNK)],
            cache_out_a.at[pl.ds(c * _PAGES_PER_CHUNK, _PAGES_PER_CHUNK)],
            sems.at[c])
        cp.start()
        copies.append(cp)

    parent = meta_s[0]
    child = meta_s[1]
    fresh = meta_s[2]

    src_len = seq_lens_s[parent]
    last_idx = (src_len + PAGE_SIZE - 1) // PAGE_SIZE - 1
    safe_last = jnp.maximum(last_idx, 0)
    has_partial = jnp.logical_and(src_len % PAGE_SIZE != 0, src_len > 0)
    src_page = pi_s[parent, safe_last]
    dst_page = jnp.where(has_partial, fresh, src_page)

    rows_t = jax.lax.broadcasted_iota(jnp.int32, (MAX_SEQS, MAX_SEQ_LEN), 0)
    parent_tok = tokens_v[pl.ds(parent, 1), :]
    tokens_out_v[...] = jnp.where(rows_t == child, parent_tok, tokens_v[...])

    rows_p = jax.lax.broadcasted_iota(jnp.int32, (MAX_SEQS, PAGES_PER_SEQ), 0)
    cols_p = jax.lax.broadcasted_iota(jnp.int32, (1, PAGES_PER_SEQ), 1)
    parent_row = pi_v[pl.ds(parent, 1), :]
    child_row = jnp.where(jnp.logical_and(has_partial, cols_p == safe_last),
                          fresh, parent_row)
    pi_out_v[...] = jnp.where(rows_p == child, child_row, pi_v[...])

    def _wr(i, carry):
        v = seq_lens_s[i]
        seq_lens_out_s[i] = jnp.where(i == child, src_len, v)
        return carry
    jax.lax.fori_loop(0, MAX_SEQS, _wr, 0)

    for cp in copies:
        cp.wait()

    fix = pltpu.make_async_copy(cache_a.at[pl.ds(src_page, 1)],
                                cache_out_a.at[pl.ds(dst_page, 1)],
                                sems.at[_NCHUNKS])
    fix.start()
    fix.wait()


def kernel(cache, tokens, seq_lens, page_indices, parent_local_id,
           child_local_id, fresh_page):
    meta = jnp.stack([jnp.asarray(parent_local_id, jnp.int32),
                      jnp.asarray(child_local_id, jnp.int32),
                      jnp.asarray(fresh_page, jnp.int32)])
    out = pl.pallas_call(
        _body,
        in_specs=[
            pl.BlockSpec(memory_space=pltpu.SMEM),
            pl.BlockSpec(memory_space=pltpu.SMEM),
            pl.BlockSpec(memory_space=pltpu.SMEM),
            pl.BlockSpec(memory_space=pltpu.VMEM),
            pl.BlockSpec(memory_space=pltpu.VMEM),
            pl.BlockSpec(memory_space=pl.ANY),
        ],
        out_specs=[
            pl.BlockSpec(memory_space=pl.ANY),
            pl.BlockSpec(memory_space=pltpu.VMEM),
            pl.BlockSpec(memory_space=pltpu.SMEM),
            pl.BlockSpec(memory_space=pltpu.VMEM),
        ],
        out_shape=[
            jax.ShapeDtypeStruct(cache.shape, cache.dtype),
            jax.ShapeDtypeStruct(tokens.shape, tokens.dtype),
            jax.ShapeDtypeStruct(seq_lens.shape, seq_lens.dtype),
            jax.ShapeDtypeStruct(page_indices.shape, page_indices.dtype),
        ],
        scratch_shapes=[pltpu.SemaphoreType.DMA((_NCHUNKS + 1,))],
    )(meta, seq_lens, page_indices, page_indices, tokens, cache)
    cache_out, tokens_out, seq_lens_out, pi_out = out
    return (cache_out, tokens_out, seq_lens_out, pi_out)


# speedup vs baseline: 39.0289x; 39.0289x over previous
import jax
import jax.numpy as jnp
from jax.experimental import pallas as pl
from jax.experimental.pallas import tpu as pltpu

NUM_PAGES = 2048
PAGE_SIZE = 64
KV_DIM = 256
MAX_SEQS = 64
PAGES_PER_SEQ = 64
MAX_SEQ_LEN = 4096

_BP = 32
_NBLK = NUM_PAGES // _BP


def _body(idx_ref, cache_blk, srcpg, meta_s, seq_lens_s, pi_s, pi_v, tokens_v,
          out_blk, tokens_out_v, seq_lens_out_s, pi_out_v):
    i = pl.program_id(0)
    out_blk[...] = cache_blk[...]

    dst_page = idx_ref[1]

    @pl.when(i == dst_page // _BP)
    def _fix():
        out_blk[pl.ds(dst_page % _BP, 1)] = srcpg[...]

    @pl.when(i == 0)
    def _metadata():
        parent = meta_s[0]
        child = meta_s[1]
        fresh = meta_s[2]

        src_len = seq_lens_s[parent]
        last_idx = (src_len + PAGE_SIZE - 1) // PAGE_SIZE - 1
        safe_last = jnp.maximum(last_idx, 0)
        has_partial = jnp.logical_and(src_len % PAGE_SIZE != 0, src_len > 0)

        rows_t = jax.lax.broadcasted_iota(jnp.int32, (MAX_SEQS, MAX_SEQ_LEN), 0)
        parent_tok = tokens_v[pl.ds(parent, 1), :]
        tokens_out_v[...] = jnp.where(rows_t == child, parent_tok, tokens_v[...])

        rows_p = jax.lax.broadcasted_iota(jnp.int32, (MAX_SEQS, PAGES_PER_SEQ), 0)
        cols_p = jax.lax.broadcasted_iota(jnp.int32, (1, PAGES_PER_SEQ), 1)
        parent_row = pi_v[pl.ds(parent, 1), :]
        child_row = jnp.where(jnp.logical_and(has_partial, cols_p == safe_last),
                              fresh, parent_row)
        pi_out_v[...] = jnp.where(rows_p == child, child_row, pi_v[...])

        def _wr(k, carry):
            v = seq_lens_s[k]
            seq_lens_out_s[k] = jnp.where(k == child, src_len, v)
            return carry
        jax.lax.fori_loop(0, MAX_SEQS, _wr, 0)


def kernel(cache, tokens, seq_lens, page_indices, parent_local_id,
           child_local_id, fresh_page):
    parent = jnp.asarray(parent_local_id, jnp.int32)
    child = jnp.asarray(child_local_id, jnp.int32)
    fresh = jnp.asarray(fresh_page, jnp.int32)

    src_len = seq_lens[parent]
    safe_last = jnp.maximum((src_len + PAGE_SIZE - 1) // PAGE_SIZE - 1, 0)
    has_partial = jnp.logical_and(src_len % PAGE_SIZE != 0, src_len > 0)
    src_page = page_indices[parent, safe_last]
    dst_page = jnp.where(has_partial, fresh, src_page)
    idx = jnp.stack([src_page, dst_page])
    meta = jnp.stack([parent, child, fresh])

    grid_spec = pltpu.PrefetchScalarGridSpec(
        num_scalar_prefetch=1,
        grid=(_NBLK,),
        in_specs=[
            pl.BlockSpec((_BP, PAGE_SIZE, KV_DIM), lambda i, idx: (i, 0, 0)),
            pl.BlockSpec((1, PAGE_SIZE, KV_DIM), lambda i, idx: (idx[0], 0, 0)),
            pl.BlockSpec(memory_space=pltpu.SMEM),
            pl.BlockSpec(memory_space=pltpu.SMEM),
            pl.BlockSpec(memory_space=pltpu.SMEM),
            pl.BlockSpec((MAX_SEQS, PAGES_PER_SEQ), lambda i, idx: (0, 0)),
            pl.BlockSpec((MAX_SEQS, MAX_SEQ_LEN), lambda i, idx: (0, 0)),
        ],
        out_specs=[
            pl.BlockSpec((_BP, PAGE_SIZE, KV_DIM), lambda i, idx: (i, 0, 0)),
            pl.BlockSpec((MAX_SEQS, MAX_SEQ_LEN), lambda i, idx: (0, 0)),
            pl.BlockSpec(memory_space=pltpu.SMEM),
            pl.BlockSpec((MAX_SEQS, PAGES_PER_SEQ), lambda i, idx: (0, 0)),
        ],
    )
    out = pl.pallas_call(
        _body,
        grid_spec=grid_spec,
        out_shape=[
            jax.ShapeDtypeStruct(cache.shape, cache.dtype),
            jax.ShapeDtypeStruct(tokens.shape, tokens.dtype),
            jax.ShapeDtypeStruct(seq_lens.shape, seq_lens.dtype),
            jax.ShapeDtypeStruct(page_indices.shape, page_indices.dtype),
        ],
        compiler_params=pltpu.CompilerParams(
            dimension_semantics=("arbitrary",)),
    )(idx, cache, cache, meta, seq_lens, page_indices, page_indices, tokens)
    cache_out, tokens_out, seq_lens_out, pi_out = out
    return (cache_out, tokens_out, seq_lens_out, pi_out)


# speedup vs baseline: 41.9922x; 1.0759x over previous
import jax
import jax.numpy as jnp
from jax.experimental import pallas as pl
from jax.experimental.pallas import tpu as pltpu

NUM_PAGES = 2048
PAGE_SIZE = 64
KV_DIM = 256
MAX_SEQS = 64
PAGES_PER_SEQ = 64
MAX_SEQ_LEN = 4096

_BP = 64
_NBLK = NUM_PAGES // _BP


def _body(idx_ref, cache_blk, srcpg, meta_s, seq_lens_s, pi_s, pi_v, tokens_v,
          out_blk, tokens_out_v, seq_lens_out_s, pi_out_v):
    i = pl.program_id(0)
    out_blk[...] = cache_blk[...]

    dst_page = idx_ref[1]

    @pl.when(i == dst_page // _BP)
    def _fix():
        out_blk[pl.ds(dst_page % _BP, 1)] = srcpg[...]

    @pl.when(i == 0)
    def _metadata():
        parent = meta_s[0]
        child = meta_s[1]
        fresh = meta_s[2]

        src_len = seq_lens_s[parent]
        last_idx = (src_len + PAGE_SIZE - 1) // PAGE_SIZE - 1
        safe_last = jnp.maximum(last_idx, 0)
        has_partial = jnp.logical_and(src_len % PAGE_SIZE != 0, src_len > 0)

        rows_t = jax.lax.broadcasted_iota(jnp.int32, (MAX_SEQS, MAX_SEQ_LEN), 0)
        parent_tok = tokens_v[pl.ds(parent, 1), :]
        tokens_out_v[...] = jnp.where(rows_t == child, parent_tok, tokens_v[...])

        rows_p = jax.lax.broadcasted_iota(jnp.int32, (MAX_SEQS, PAGES_PER_SEQ), 0)
        cols_p = jax.lax.broadcasted_iota(jnp.int32, (1, PAGES_PER_SEQ), 1)
        parent_row = pi_v[pl.ds(parent, 1), :]
        child_row = jnp.where(jnp.logical_and(has_partial, cols_p == safe_last),
                              fresh, parent_row)
        pi_out_v[...] = jnp.where(rows_p == child, child_row, pi_v[...])

        def _wr(k, carry):
            v = seq_lens_s[k]
            seq_lens_out_s[k] = jnp.where(k == child, src_len, v)
            return carry
        jax.lax.fori_loop(0, MAX_SEQS, _wr, 0)


def kernel(cache, tokens, seq_lens, page_indices, parent_local_id,
           child_local_id, fresh_page):
    parent = jnp.asarray(parent_local_id, jnp.int32)
    child = jnp.asarray(child_local_id, jnp.int32)
    fresh = jnp.asarray(fresh_page, jnp.int32)

    src_len = seq_lens[parent]
    safe_last = jnp.maximum((src_len + PAGE_SIZE - 1) // PAGE_SIZE - 1, 0)
    has_partial = jnp.logical_and(src_len % PAGE_SIZE != 0, src_len > 0)
    src_page = page_indices[parent, safe_last]
    dst_page = jnp.where(has_partial, fresh, src_page)
    idx = jnp.stack([src_page, dst_page])
    meta = jnp.stack([parent, child, fresh])

    grid_spec = pltpu.PrefetchScalarGridSpec(
        num_scalar_prefetch=1,
        grid=(_NBLK,),
        in_specs=[
            pl.BlockSpec((_BP, PAGE_SIZE, KV_DIM), lambda i, idx: (i, 0, 0)),
            pl.BlockSpec((1, PAGE_SIZE, KV_DIM), lambda i, idx: (idx[0], 0, 0)),
            pl.BlockSpec(memory_space=pltpu.SMEM),
            pl.BlockSpec(memory_space=pltpu.SMEM),
            pl.BlockSpec(memory_space=pltpu.SMEM),
            pl.BlockSpec((MAX_SEQS, PAGES_PER_SEQ), lambda i, idx: (0, 0)),
            pl.BlockSpec((MAX_SEQS, MAX_SEQ_LEN), lambda i, idx: (0, 0)),
        ],
        out_specs=[
            pl.BlockSpec((_BP, PAGE_SIZE, KV_DIM), lambda i, idx: (i, 0, 0)),
            pl.BlockSpec((MAX_SEQS, MAX_SEQ_LEN), lambda i, idx: (0, 0)),
            pl.BlockSpec(memory_space=pltpu.SMEM),
            pl.BlockSpec((MAX_SEQS, PAGES_PER_SEQ), lambda i, idx: (0, 0)),
        ],
    )
    out = pl.pallas_call(
        _body,
        grid_spec=grid_spec,
        out_shape=[
            jax.ShapeDtypeStruct(cache.shape, cache.dtype),
            jax.ShapeDtypeStruct(tokens.shape, tokens.dtype),
            jax.ShapeDtypeStruct(seq_lens.shape, seq_lens.dtype),
            jax.ShapeDtypeStruct(page_indices.shape, page_indices.dtype),
        ],
        compiler_params=pltpu.CompilerParams(
            dimension_semantics=("arbitrary",)),
    )(idx, cache, cache, meta, seq_lens, page_indices, page_indices, tokens)
    cache_out, tokens_out, seq_lens_out, pi_out = out
    return (cache_out, tokens_out, seq_lens_out, pi_out)


# speedup vs baseline: 42.8774x; 1.0211x over previous
import jax
import jax.numpy as jnp
from jax.experimental import pallas as pl
from jax.experimental.pallas import tpu as pltpu

NUM_PAGES = 2048
PAGE_SIZE = 64
KV_DIM = 256
MAX_SEQS = 64
PAGES_PER_SEQ = 64
MAX_SEQ_LEN = 4096

_BP = 128
_NBLK = NUM_PAGES // _BP


def _body(idx_ref, cache_blk, srcpg, meta_s, seq_lens_s, pi_s, pi_v, tokens_v,
          out_blk, tokens_out_v, seq_lens_out_s, pi_out_v):
    i = pl.program_id(0)
    out_blk[...] = cache_blk[...]

    dst_page = idx_ref[1]

    @pl.when(i == dst_page // _BP)
    def _fix():
        out_blk[pl.ds(dst_page % _BP, 1)] = srcpg[...]

    @pl.when(i == 0)
    def _metadata():
        parent = meta_s[0]
        child = meta_s[1]
        fresh = meta_s[2]

        src_len = seq_lens_s[parent]
        last_idx = (src_len + PAGE_SIZE - 1) // PAGE_SIZE - 1
        safe_last = jnp.maximum(last_idx, 0)
        has_partial = jnp.logical_and(src_len % PAGE_SIZE != 0, src_len > 0)

        rows_t = jax.lax.broadcasted_iota(jnp.int32, (MAX_SEQS, MAX_SEQ_LEN), 0)
        parent_tok = tokens_v[pl.ds(parent, 1), :]
        tokens_out_v[...] = jnp.where(rows_t == child, parent_tok, tokens_v[...])

        rows_p = jax.lax.broadcasted_iota(jnp.int32, (MAX_SEQS, PAGES_PER_SEQ), 0)
        cols_p = jax.lax.broadcasted_iota(jnp.int32, (1, PAGES_PER_SEQ), 1)
        parent_row = pi_v[pl.ds(parent, 1), :]
        child_row = jnp.where(jnp.logical_and(has_partial, cols_p == safe_last),
                              fresh, parent_row)
        pi_out_v[...] = jnp.where(rows_p == child, child_row, pi_v[...])

        def _wr(k, carry):
            v = seq_lens_s[k]
            seq_lens_out_s[k] = jnp.where(k == child, src_len, v)
            return carry
        jax.lax.fori_loop(0, MAX_SEQS, _wr, 0)


def kernel(cache, tokens, seq_lens, page_indices, parent_local_id,
           child_local_id, fresh_page):
    parent = jnp.asarray(parent_local_id, jnp.int32)
    child = jnp.asarray(child_local_id, jnp.int32)
    fresh = jnp.asarray(fresh_page, jnp.int32)

    src_len = seq_lens[parent]
    safe_last = jnp.maximum((src_len + PAGE_SIZE - 1) // PAGE_SIZE - 1, 0)
    has_partial = jnp.logical_and(src_len % PAGE_SIZE != 0, src_len > 0)
    src_page = page_indices[parent, safe_last]
    dst_page = jnp.where(has_partial, fresh, src_page)
    idx = jnp.stack([src_page, dst_page])
    meta = jnp.stack([parent, child, fresh])

    grid_spec = pltpu.PrefetchScalarGridSpec(
        num_scalar_prefetch=1,
        grid=(_NBLK,),
        in_specs=[
            pl.BlockSpec((_BP, PAGE_SIZE, KV_DIM), lambda i, idx: (i, 0, 0)),
            pl.BlockSpec((1, PAGE_SIZE, KV_DIM), lambda i, idx: (idx[0], 0, 0)),
            pl.BlockSpec(memory_space=pltpu.SMEM),
            pl.BlockSpec(memory_space=pltpu.SMEM),
            pl.BlockSpec(memory_space=pltpu.SMEM),
            pl.BlockSpec((MAX_SEQS, PAGES_PER_SEQ), lambda i, idx: (0, 0)),
            pl.BlockSpec((MAX_SEQS, MAX_SEQ_LEN), lambda i, idx: (0, 0)),
        ],
        out_specs=[
            pl.BlockSpec((_BP, PAGE_SIZE, KV_DIM), lambda i, idx: (i, 0, 0)),
            pl.BlockSpec((MAX_SEQS, MAX_SEQ_LEN), lambda i, idx: (0, 0)),
            pl.BlockSpec(memory_space=pltpu.SMEM),
            pl.BlockSpec((MAX_SEQS, PAGES_PER_SEQ), lambda i, idx: (0, 0)),
        ],
    )
    out = pl.pallas_call(
        _body,
        grid_spec=grid_spec,
        out_shape=[
            jax.ShapeDtypeStruct(cache.shape, cache.dtype),
            jax.ShapeDtypeStruct(tokens.shape, tokens.dtype),
            jax.ShapeDtypeStruct(seq_lens.shape, seq_lens.dtype),
            jax.ShapeDtypeStruct(page_indices.shape, page_indices.dtype),
        ],
        compiler_params=pltpu.CompilerParams(
            dimension_semantics=("arbitrary",)),
    )(idx, cache, cache, meta, seq_lens, page_indices, page_indices, tokens)
    cache_out, tokens_out, seq_lens_out, pi_out = out
    return (cache_out, tokens_out, seq_lens_out, pi_out)


# speedup vs baseline: 45.0694x; 1.0511x over previous
import jax
import jax.numpy as jnp
from jax.experimental import pallas as pl
from jax.experimental.pallas import tpu as pltpu

NUM_PAGES = 2048
PAGE_SIZE = 64
KV_DIM = 256
MAX_SEQS = 64
PAGES_PER_SEQ = 64
MAX_SEQ_LEN = 4096

_BP = 16
_NCH = NUM_PAGES // _BP
_L = 4
_M = 4
_K = _L + _M


def _body(meta_s, seq_lens_s, pi_s, pi_v, tokens_v, cache_a,
          cache_out_a, tokens_out_v, seq_lens_out_s, pi_out_v,
          bufs, pbuf, in_sems, out_sems, psem):

    def in_copy(c):
        k = jax.lax.rem(c, _K)
        return pltpu.make_async_copy(
            cache_a.at[pl.ds(c * _BP, _BP)], bufs.at[k], in_sems.at[k])

    def out_copy(c):
        k = jax.lax.rem(c, _K)
        return pltpu.make_async_copy(
            bufs.at[k], cache_out_a.at[pl.ds(c * _BP, _BP)], out_sems.at[k])

    for c in range(_L):
        in_copy(c).start()

    def step(c, carry):
        @pl.when(c >= _M)
        def _():
            out_copy(c - _M).wait()

        @pl.when(c + _L < _NCH)
        def _():
            in_copy(c + _L).start()

        in_copy(c).wait()
        out_copy(c).start()
        return carry

    jax.lax.fori_loop(0, _NCH, step, 0)

    parent = meta_s[0]
    child = meta_s[1]
    fresh = meta_s[2]

    src_len = seq_lens_s[parent]
    last_idx = (src_len + PAGE_SIZE - 1) // PAGE_SIZE - 1
    safe_last = jnp.maximum(last_idx, 0)
    has_partial = jnp.logical_and(src_len % PAGE_SIZE != 0, src_len > 0)
    src_page = pi_s[parent, safe_last]
    dst_page = jnp.where(has_partial, fresh, src_page)

    rows_t = jax.lax.broadcasted_iota(jnp.int32, (MAX_SEQS, MAX_SEQ_LEN), 0)
    parent_tok = tokens_v[pl.ds(parent, 1), :]
    tokens_out_v[...] = jnp.where(rows_t == child, parent_tok, tokens_v[...])

    rows_p = jax.lax.broadcasted_iota(jnp.int32, (MAX_SEQS, PAGES_PER_SEQ), 0)
    cols_p = jax.lax.broadcasted_iota(jnp.int32, (1, PAGES_PER_SEQ), 1)
    parent_row = pi_v[pl.ds(parent, 1), :]
    child_row = jnp.where(jnp.logical_and(has_partial, cols_p == safe_last),
                          fresh, parent_row)
    pi_out_v[...] = jnp.where(rows_p == child, child_row, pi_v[...])

    def _wr(k, carry):
        v = seq_lens_s[k]
        seq_lens_out_s[k] = jnp.where(k == child, src_len, v)
        return carry
    jax.lax.fori_loop(0, MAX_SEQS, _wr, 0)

    def drain(c, carry):
        out_copy(c).wait()
        return carry
    jax.lax.fori_loop(max(_NCH - _M, 0), _NCH, drain, 0)

    fin = pltpu.make_async_copy(cache_a.at[pl.ds(src_page, 1)], pbuf, psem)
    fin.start()
    fin.wait()
    fout = pltpu.make_async_copy(pbuf, cache_out_a.at[pl.ds(dst_page, 1)], psem)
    fout.start()
    fout.wait()


def kernel(cache, tokens, seq_lens, page_indices, parent_local_id,
           child_local_id, fresh_page):
    meta = jnp.stack([jnp.asarray(parent_local_id, jnp.int32),
                      jnp.asarray(child_local_id, jnp.int32),
                      jnp.asarray(fresh_page, jnp.int32)])
    out = pl.pallas_call(
        _body,
        in_specs=[
            pl.BlockSpec(memory_space=pltpu.SMEM),
            pl.BlockSpec(memory_space=pltpu.SMEM),
            pl.BlockSpec(memory_space=pltpu.SMEM),
            pl.BlockSpec(memory_space=pltpu.VMEM),
            pl.BlockSpec(memory_space=pltpu.VMEM),
            pl.BlockSpec(memory_space=pl.ANY),
        ],
        out_specs=[
            pl.BlockSpec(memory_space=pl.ANY),
            pl.BlockSpec(memory_space=pltpu.VMEM),
            pl.BlockSpec(memory_space=pltpu.SMEM),
            pl.BlockSpec(memory_space=pltpu.VMEM),
        ],
        out_shape=[
            jax.ShapeDtypeStruct(cache.shape, cache.dtype),
            jax.ShapeDtypeStruct(tokens.shape, tokens.dtype),
            jax.ShapeDtypeStruct(seq_lens.shape, seq_lens.dtype),
            jax.ShapeDtypeStruct(page_indices.shape, page_indices.dtype),
        ],
        scratch_shapes=[
            pltpu.VMEM((_K, _BP, PAGE_SIZE, KV_DIM), jnp.float32),
            pltpu.VMEM((1, PAGE_SIZE, KV_DIM), jnp.float32),
            pltpu.SemaphoreType.DMA((_K,)),
            pltpu.SemaphoreType.DMA((_K,)),
            pltpu.SemaphoreType.DMA,
        ],
        compiler_params=pltpu.CompilerParams(
            vmem_limit_bytes=128 * 1024 * 1024),
    )(meta, seq_lens, page_indices, page_indices, tokens, cache)
    cache_out, tokens_out, seq_lens_out, pi_out = out
    return (cache_out, tokens_out, seq_lens_out, pi_out)


# speedup vs baseline: 45.2840x; 1.0048x over previous
import jax
import jax.numpy as jnp
from jax.experimental import pallas as pl
from jax.experimental.pallas import tpu as pltpu

NUM_PAGES = 2048
PAGE_SIZE = 64
KV_DIM = 256
MAX_SEQS = 64
PAGES_PER_SEQ = 64
MAX_SEQ_LEN = 4096

_BP = 16
_NCH = NUM_PAGES // _BP
_L = 8
_M = 8
_K = _L + _M


def _body(meta_s, seq_lens_s, pi_s, pi_v, tokens_v, cache_a,
          cache_out_a, tokens_out_v, seq_lens_out_s, pi_out_v,
          bufs, pbuf, in_sems, out_sems, psem):

    def in_copy(c):
        k = jax.lax.rem(c, _K)
        return pltpu.make_async_copy(
            cache_a.at[pl.ds(c * _BP, _BP)], bufs.at[k], in_sems.at[k])

    def out_copy(c):
        k = jax.lax.rem(c, _K)
        return pltpu.make_async_copy(
            bufs.at[k], cache_out_a.at[pl.ds(c * _BP, _BP)], out_sems.at[k])

    for c in range(_L):
        in_copy(c).start()

    def step(c, carry):
        @pl.when(c >= _M)
        def _():
            out_copy(c - _M).wait()

        @pl.when(c + _L < _NCH)
        def _():
            in_copy(c + _L).start()

        in_copy(c).wait()
        out_copy(c).start()
        return carry

    jax.lax.fori_loop(0, _NCH, step, 0)

    parent = meta_s[0]
    child = meta_s[1]
    fresh = meta_s[2]

    src_len = seq_lens_s[parent]
    last_idx = (src_len + PAGE_SIZE - 1) // PAGE_SIZE - 1
    safe_last = jnp.maximum(last_idx, 0)
    has_partial = jnp.logical_and(src_len % PAGE_SIZE != 0, src_len > 0)
    src_page = pi_s[parent, safe_last]
    dst_page = jnp.where(has_partial, fresh, src_page)

    rows_t = jax.lax.broadcasted_iota(jnp.int32, (MAX_SEQS, MAX_SEQ_LEN), 0)
    parent_tok = tokens_v[pl.ds(parent, 1), :]
    tokens_out_v[...] = jnp.where(rows_t == child, parent_tok, tokens_v[...])

    rows_p = jax.lax.broadcasted_iota(jnp.int32, (MAX_SEQS, PAGES_PER_SEQ), 0)
    cols_p = jax.lax.broadcasted_iota(jnp.int32, (1, PAGES_PER_SEQ), 1)
    parent_row = pi_v[pl.ds(parent, 1), :]
    child_row = jnp.where(jnp.logical_and(has_partial, cols_p == safe_last),
                          fresh, parent_row)
    pi_out_v[...] = jnp.where(rows_p == child, child_row, pi_v[...])

    def _wr(k, carry):
        v = seq_lens_s[k]
        seq_lens_out_s[k] = jnp.where(k == child, src_len, v)
        return carry
    jax.lax.fori_loop(0, MAX_SEQS, _wr, 0)

    def drain(c, carry):
        out_copy(c).wait()
        return carry
    jax.lax.fori_loop(max(_NCH - _M, 0), _NCH, drain, 0)

    fin = pltpu.make_async_copy(cache_a.at[pl.ds(src_page, 1)], pbuf, psem)
    fin.start()
    fin.wait()
    fout = pltpu.make_async_copy(pbuf, cache_out_a.at[pl.ds(dst_page, 1)], psem)
    fout.start()
    fout.wait()


def kernel(cache, tokens, seq_lens, page_indices, parent_local_id,
           child_local_id, fresh_page):
    meta = jnp.stack([jnp.asarray(parent_local_id, jnp.int32),
                      jnp.asarray(child_local_id, jnp.int32),
                      jnp.asarray(fresh_page, jnp.int32)])
    out = pl.pallas_call(
        _body,
        in_specs=[
            pl.BlockSpec(memory_space=pltpu.SMEM),
            pl.BlockSpec(memory_space=pltpu.SMEM),
            pl.BlockSpec(memory_space=pltpu.SMEM),
            pl.BlockSpec(memory_space=pltpu.VMEM),
            pl.BlockSpec(memory_space=pltpu.VMEM),
            pl.BlockSpec(memory_space=pl.ANY),
        ],
        out_specs=[
            pl.BlockSpec(memory_space=pl.ANY),
            pl.BlockSpec(memory_space=pltpu.VMEM),
            pl.BlockSpec(memory_space=pltpu.SMEM),
            pl.BlockSpec(memory_space=pltpu.VMEM),
        ],
        out_shape=[
            jax.ShapeDtypeStruct(cache.shape, cache.dtype),
            jax.ShapeDtypeStruct(tokens.shape, tokens.dtype),
            jax.ShapeDtypeStruct(seq_lens.shape, seq_lens.dtype),
            jax.ShapeDtypeStruct(page_indices.shape, page_indices.dtype),
        ],
        scratch_shapes=[
            pltpu.VMEM((_K, _BP, PAGE_SIZE, KV_DIM), jnp.float32),
            pltpu.VMEM((1, PAGE_SIZE, KV_DIM), jnp.float32),
            pltpu.SemaphoreType.DMA((_K,)),
            pltpu.SemaphoreType.DMA((_K,)),
            pltpu.SemaphoreType.DMA,
        ],
        compiler_params=pltpu.CompilerParams(
            vmem_limit_bytes=128 * 1024 * 1024),
    )(meta, seq_lens, page_indices, page_indices, tokens, cache)
    cache_out, tokens_out, seq_lens_out, pi_out = out
    return (cache_out, tokens_out, seq_lens_out, pi_out)


# speedup vs baseline: 45.3355x; 1.0011x over previous
import jax
import jax.numpy as jnp
from jax.experimental import pallas as pl
from jax.experimental.pallas import tpu as pltpu

NUM_PAGES = 2048
PAGE_SIZE = 64
KV_DIM = 256
MAX_SEQS = 64
PAGES_PER_SEQ = 64
MAX_SEQ_LEN = 4096

_BP = 32
_NCH = NUM_PAGES // _BP
_L = 8
_M = 8
_K = _L + _M


def _body(meta_s, seq_lens_s, pi_s, pi_v, tokens_v, cache_a,
          cache_out_a, tokens_out_v, seq_lens_out_s, pi_out_v,
          bufs, pbuf, in_sems, out_sems, psem):

    def in_copy(c):
        k = jax.lax.rem(c, _K)
        return pltpu.make_async_copy(
            cache_a.at[pl.ds(c * _BP, _BP)], bufs.at[k], in_sems.at[k])

    def out_copy(c):
        k = jax.lax.rem(c, _K)
        return pltpu.make_async_copy(
            bufs.at[k], cache_out_a.at[pl.ds(c * _BP, _BP)], out_sems.at[k])

    for c in range(_L):
        in_copy(c).start()

    def step(c, carry):
        @pl.when(c >= _M)
        def _():
            out_copy(c - _M).wait()

        @pl.when(c + _L < _NCH)
        def _():
            in_copy(c + _L).start()

        in_copy(c).wait()
        out_copy(c).start()
        return carry

    jax.lax.fori_loop(0, _NCH, step, 0)

    parent = meta_s[0]
    child = meta_s[1]
    fresh = meta_s[2]

    src_len = seq_lens_s[parent]
    last_idx = (src_len + PAGE_SIZE - 1) // PAGE_SIZE - 1
    safe_last = jnp.maximum(last_idx, 0)
    has_partial = jnp.logical_and(src_len % PAGE_SIZE != 0, src_len > 0)
    src_page = pi_s[parent, safe_last]
    dst_page = jnp.where(has_partial, fresh, src_page)

    rows_t = jax.lax.broadcasted_iota(jnp.int32, (MAX_SEQS, MAX_SEQ_LEN), 0)
    parent_tok = tokens_v[pl.ds(parent, 1), :]
    tokens_out_v[...] = jnp.where(rows_t == child, parent_tok, tokens_v[...])

    rows_p = jax.lax.broadcasted_iota(jnp.int32, (MAX_SEQS, PAGES_PER_SEQ), 0)
    cols_p = jax.lax.broadcasted_iota(jnp.int32, (1, PAGES_PER_SEQ), 1)
    parent_row = pi_v[pl.ds(parent, 1), :]
    child_row = jnp.where(jnp.logical_and(has_partial, cols_p == safe_last),
                          fresh, parent_row)
    pi_out_v[...] = jnp.where(rows_p == child, child_row, pi_v[...])

    def _wr(k, carry):
        v = seq_lens_s[k]
        seq_lens_out_s[k] = jnp.where(k == child, src_len, v)
        return carry
    jax.lax.fori_loop(0, MAX_SEQS, _wr, 0)

    def drain(c, carry):
        out_copy(c).wait()
        return carry
    jax.lax.fori_loop(max(_NCH - _M, 0), _NCH, drain, 0)

    fin = pltpu.make_async_copy(cache_a.at[pl.ds(src_page, 1)], pbuf, psem)
    fin.start()
    fin.wait()
    fout = pltpu.make_async_copy(pbuf, cache_out_a.at[pl.ds(dst_page, 1)], psem)
    fout.start()
    fout.wait()


def kernel(cache, tokens, seq_lens, page_indices, parent_local_id,
           child_local_id, fresh_page):
    meta = jnp.stack([jnp.asarray(parent_local_id, jnp.int32),
                      jnp.asarray(child_local_id, jnp.int32),
                      jnp.asarray(fresh_page, jnp.int32)])
    out = pl.pallas_call(
        _body,
        in_specs=[
            pl.BlockSpec(memory_space=pltpu.SMEM),
            pl.BlockSpec(memory_space=pltpu.SMEM),
            pl.BlockSpec(memory_space=pltpu.SMEM),
            pl.BlockSpec(memory_space=pltpu.VMEM),
            pl.BlockSpec(memory_space=pltpu.VMEM),
            pl.BlockSpec(memory_space=pl.ANY),
        ],
        out_specs=[
            pl.BlockSpec(memory_space=pl.ANY),
            pl.BlockSpec(memory_space=pltpu.VMEM),
            pl.BlockSpec(memory_space=pltpu.SMEM),
            pl.BlockSpec(memory_space=pltpu.VMEM),
        ],
        out_shape=[
            jax.ShapeDtypeStruct(cache.shape, cache.dtype),
            jax.ShapeDtypeStruct(tokens.shape, tokens.dtype),
            jax.ShapeDtypeStruct(seq_lens.shape, seq_lens.dtype),
            jax.ShapeDtypeStruct(page_indices.shape, page_indices.dtype),
        ],
        scratch_shapes=[
            pltpu.VMEM((_K, _BP, PAGE_SIZE, KV_DIM), jnp.float32),
            pltpu.VMEM((1, PAGE_SIZE, KV_DIM), jnp.float32),
            pltpu.SemaphoreType.DMA((_K,)),
            pltpu.SemaphoreType.DMA((_K,)),
            pltpu.SemaphoreType.DMA,
        ],
        compiler_params=pltpu.CompilerParams(
            vmem_limit_bytes=128 * 1024 * 1024),
    )(meta, seq_lens, page_indices, page_indices, tokens, cache)
    cache_out, tokens_out, seq_lens_out, pi_out = out
    return (cache_out, tokens_out, seq_lens_out, pi_out)
